# baseline (speedup 1.0000x reference)
import functools

import jax
import jax.numpy as jnp
from jax import lax
from jax.experimental import pallas as pl
from jax.experimental.pallas import tpu as pltpu
from jax.experimental.pallas import tpu_sc as plsc

_R, _C = 1024, 128
_N = _R * _C
_LOGN = 17


_BIAS = -2147483648


def _pos_iotas():
    row = jax.lax.broadcasted_iota(jnp.int32, (_R, _C), 0)
    lane = jax.lax.broadcasted_iota(jnp.int32, (_R, _C), 1)
    return row, lane


def _partner_sel(d, row, lane):
    if d < _C:
        return 1, d, (lane & d) == 0
    return 0, d // _C, (row & (d // _C)) == 0


def _xor_shuffle(x, axis, dist, low):
    if axis == 0 and dist >= 8:
        parts = []
        for g in range(_R // (2 * dist)):
            parts.append(x[(2 * g + 1) * dist:(2 * g + 2) * dist])
            parts.append(x[(2 * g) * dist:(2 * g + 1) * dist])
        return jnp.concatenate(parts, axis=0)
    fwd = pltpu.roll(x, x.shape[axis] - dist, axis)
    bwd = pltpu.roll(x, dist, axis)
    return jnp.where(low, fwd, bwd)


def _body(vt_ref, ot_ref, mf_ref, groups_ref, sidx_ref, total_ref):
    row, lane = _pos_iotas()

    acc = jnp.zeros((_R, _C), jnp.float32)
    for k in range(5):
        dx = vt_ref[3 * k + 3] - vt_ref[3 * k]
        dy = vt_ref[3 * k + 4] - vt_ref[3 * k + 1]
        dz = vt_ref[3 * k + 5] - vt_ref[3 * k + 2]
        acc = acc + jnp.sqrt(dx * dx + dy * dy + dz * dz)
    total_ref[0, 0] = jnp.sum(acc * mf_ref[...])

    o0, o1, o2 = ot_ref[0], ot_ref[1], ot_ref[2]
    o3, o4, o5 = ot_ref[3], ot_ref[4], ot_ref[5]
    khi = ((o0 * 200 + o1) * 200 + o2) * 200 + o3
    klo_s = o4 * 200 + o5
    idx = row * _C + lane

    for k in range(1, _LOGN + 1):
        if k < 7:
            asc = (lane & (1 << k)) == 0
        elif k < _LOGN:
            asc = (row & (1 << (k - 7))) == 0
        else:
            asc = None
        for j in range(k - 1, -1, -1):
            d = 1 << j
            axis, dist, low = _partner_sel(d, row, lane)
            phi = _xor_shuffle(khi, axis, dist, low)
            plo = _xor_shuffle(klo_s, axis, dist, low)
            pidx = _xor_shuffle(idx, axis, dist, low)
            lt = (khi < phi) | ((khi == phi) & (klo_s < plo))
            eq = (khi == phi) & (klo_s == plo)
            cond = low if asc is None else ~(low ^ asc)
            keep = (cond & (lt | eq)) | (~cond & ~lt)
            khi = jnp.where(keep, khi, phi)
            klo_s = jnp.where(keep, klo_s, plo)
            idx = jnp.where(keep, idx, pidx)

    def prev_flat(x):
        rl = pltpu.roll(x, 1, 1)
        rl2 = pltpu.roll(rl, 1, 0)
        return jnp.where(lane == 0, rl2, rl)

    flag = (khi != prev_flat(khi)) | (klo_s != prev_flat(klo_s))
    flag = flag & ~((row == 0) & (lane == 0))
    rank = flag.astype(jnp.int32)
    for s in (1, 2, 4, 8, 16, 32, 64):
        rank = rank + jnp.where(lane >= s, pltpu.roll(rank, s, 1), 0)
    rowtot = rank[:, _C - 1:_C]
    rcol = jax.lax.broadcasted_iota(jnp.int32, (_R, 1), 0)
    pref = jnp.where(rcol >= 1, pltpu.roll(rowtot, 1, 0), 0)
    for s in (1, 2, 4, 8, 16, 32, 64, 128, 256, 512):
        pref = pref + jnp.where(rcol >= s, pltpu.roll(pref, s, 0), 0)
    rank = rank + pref

    groups_ref[...] = rank
    sidx_ref[...] = idx


_NW = 32
_WROWS = _R // _NW


_TROWS = _R // 16


@functools.lru_cache(maxsize=1)
def _get_sc_unsort():
    @functools.partial(
        pl.kernel,
        mesh=plsc.VectorSubcoreMesh(core_axis_name="c", subcore_axis_name="s"),
        out_type=jax.ShapeDtypeStruct((_N,), jnp.int32),
        scratch_types=[
            pltpu.VMEM((_TROWS * _C,), jnp.int32),
            pltpu.VMEM((_TROWS * _C,), jnp.int32),
            pltpu.VMEM_SHARED((_N,), jnp.int32),
            pltpu.SemaphoreType.DMA,
        ],
    )
    def _sc_unsort(idx_hbm, val_hbm, out_hbm, idx_v, val_v, shared, sem):
        s = lax.axis_index("s")
        c = lax.axis_index("c")
        base = s * (_TROWS * _C)
        pltpu.sync_copy(idx_hbm.at[pl.ds(base, _TROWS * _C)], idx_v)
        pltpu.sync_copy(val_hbm.at[pl.ds(base, _TROWS * _C)], val_v)
        pltpu.sync_copy(val_v, shared.at[idx_v])
        plsc.subcore_barrier()
        off = c * (_N // 2) + s * (_N // 32)
        pltpu.sync_copy(shared.at[pl.ds(off, _N // 32)],
                        out_hbm.at[pl.ds(off, _N // 32)])

    return _sc_unsort


def kernel(vertices, objects, mask):
    vt = vertices.reshape(_N, 18).T.reshape(18, _R, _C)
    ot = objects.reshape(_N, 6).T.reshape(6, _R, _C)
    mf = mask.reshape(_R, _C).astype(jnp.float32)
    rank2d, sidx2d, total = pl.pallas_call(
        _body,
        out_shape=(
            jax.ShapeDtypeStruct((_R, _C), jnp.int32),
            jax.ShapeDtypeStruct((_R, _C), jnp.int32),
            jax.ShapeDtypeStruct((1, 1), jnp.float32),
        ),
        out_specs=(
            pl.BlockSpec(memory_space=pltpu.VMEM),
            pl.BlockSpec(memory_space=pltpu.VMEM),
            pl.BlockSpec(memory_space=pltpu.SMEM),
        ),
    )(vt, ot, mf)
    groups = _get_sc_unsort()(sidx2d.reshape(_N), rank2d.reshape(_N))
    return groups.reshape(mask.shape), total[0, 0]


# speedup vs baseline: 1.4360x; 1.4360x over previous
import functools

import jax
import jax.numpy as jnp
from jax import lax
from jax.experimental import pallas as pl
from jax.experimental.pallas import tpu as pltpu
from jax.experimental.pallas import tpu_sc as plsc

_R, _C = 1024, 128
_N = _R * _C
_LOGN = 17


_BIAS = -2147483648


def _pos_iotas():
    row = jax.lax.broadcasted_iota(jnp.int32, (_R, _C), 0)
    lane = jax.lax.broadcasted_iota(jnp.int32, (_R, _C), 1)
    return row, lane


def _partner_sel(d, row, lane):
    if d < _C:
        return 1, d, (lane & d) == 0
    return 0, d // _C, (row & (d // _C)) == 0


def _xor_shuffle(x, axis, dist, low):
    if axis == 0 and dist >= 8:
        parts = []
        for g in range(_R // (2 * dist)):
            parts.append(x[(2 * g + 1) * dist:(2 * g + 2) * dist])
            parts.append(x[(2 * g) * dist:(2 * g + 1) * dist])
        return jnp.concatenate(parts, axis=0)
    fwd = pltpu.roll(x, x.shape[axis] - dist, axis)
    bwd = pltpu.roll(x, dist, axis)
    return jnp.where(low, fwd, bwd)


def _reduce_body(vt_ref, mf_ref, total_ref):
    acc = jnp.zeros((_R, _C), jnp.float32)
    for k in range(5):
        dx = vt_ref[3 * k + 3] - vt_ref[3 * k]
        dy = vt_ref[3 * k + 4] - vt_ref[3 * k + 1]
        dz = vt_ref[3 * k + 5] - vt_ref[3 * k + 2]
        acc = acc + jnp.sqrt(dx * dx + dy * dy + dz * dz)
    total_ref[0, 0] = jnp.sum(acc * mf_ref[...])


def _body(ot_ref, groups_ref, sidx_ref):
    row, lane = _pos_iotas()

    o0, o1, o2 = ot_ref[0], ot_ref[1], ot_ref[2]
    o3, o4, o5 = ot_ref[3], ot_ref[4], ot_ref[5]
    khi = ((o0 * 200 + o1) * 200 + o2) * 200 + o3
    klo = o4 * 200 + o5
    idxl = (row & 511) * _C + lane
    kb = (klo * 65536 + idxl) ^ _BIAS

    for k in range(1, _LOGN):
        if k < 7:
            asc = (lane & (1 << k)) == 0
        else:
            asc = (row & (1 << (k - 7))) == 0
        for j in range(k - 1, -1, -1):
            d = 1 << j
            axis, dist, low = _partner_sel(d, row, lane)
            phi = _xor_shuffle(khi, axis, dist, low)
            pkb = _xor_shuffle(kb, axis, dist, low)
            lt = (khi < phi) | ((khi == phi) & (kb < pkb))
            keep = lt ^ low ^ asc
            khi = jnp.where(keep, khi, phi)
            kb = jnp.where(keep, kb, pkb)

    half = (row >= 512).astype(jnp.int32)
    for j in range(_LOGN - 1, -1, -1):
        d = 1 << j
        axis, dist, low = _partner_sel(d, row, lane)
        phi = _xor_shuffle(khi, axis, dist, low)
        pkb = _xor_shuffle(kb, axis, dist, low)
        ph = _xor_shuffle(half, axis, dist, low)
        hieq = khi == phi
        lt = (khi < phi) | (hieq & (kb < pkb))
        eq = hieq & (kb == pkb)
        keep = (low & (lt | eq)) | ~(low | lt)
        khi = jnp.where(keep, khi, phi)
        kb = jnp.where(keep, kb, pkb)
        half = jnp.where(keep, half, ph)

    kbu = kb ^ _BIAS
    klo_s = jax.lax.shift_right_logical(kbu, 16)
    idx = (kbu & 65535) + half * 65536

    def prev_flat(x):
        rl = pltpu.roll(x, 1, 1)
        rl2 = pltpu.roll(rl, 1, 0)
        return jnp.where(lane == 0, rl2, rl)

    flag = (khi != prev_flat(khi)) | (klo_s != prev_flat(klo_s))
    flag = flag & ~((row == 0) & (lane == 0))
    rank = flag.astype(jnp.int32)
    for s in (1, 2, 4, 8, 16, 32, 64):
        rank = rank + jnp.where(lane >= s, pltpu.roll(rank, s, 1), 0)
    rowtot = rank[:, _C - 1:_C]
    rcol = jax.lax.broadcasted_iota(jnp.int32, (_R, 1), 0)
    pref = jnp.where(rcol >= 1, pltpu.roll(rowtot, 1, 0), 0)
    for s in (1, 2, 4, 8, 16, 32, 64, 128, 256, 512):
        pref = pref + jnp.where(rcol >= s, pltpu.roll(pref, s, 0), 0)
    rank = rank + pref

    groups_ref[...] = rank
    sidx_ref[...] = idx


_NW = 32
_WROWS = _R // _NW


_TROWS = _R // 16


@functools.lru_cache(maxsize=1)
def _get_sc_unsort():
    @functools.partial(
        pl.kernel,
        mesh=plsc.VectorSubcoreMesh(core_axis_name="c", subcore_axis_name="s"),
        out_type=jax.ShapeDtypeStruct((_N,), jnp.int32),
        scratch_types=[
            pltpu.VMEM((_TROWS * _C,), jnp.int32),
            pltpu.VMEM((_TROWS * _C,), jnp.int32),
            pltpu.VMEM_SHARED((_N,), jnp.int32),
            pltpu.SemaphoreType.DMA,
        ],
    )
    def _sc_unsort(idx_hbm, val_hbm, out_hbm, idx_v, val_v, shared, sem):
        s = lax.axis_index("s")
        c = lax.axis_index("c")
        base = s * (_TROWS * _C)
        pltpu.sync_copy(idx_hbm.at[pl.ds(base, _TROWS * _C)], idx_v)
        pltpu.sync_copy(val_hbm.at[pl.ds(base, _TROWS * _C)], val_v)
        pltpu.sync_copy(val_v, shared.at[idx_v])
        plsc.subcore_barrier()
        off = c * (_N // 2) + s * (_N // 32)
        pltpu.sync_copy(shared.at[pl.ds(off, _N // 32)],
                        out_hbm.at[pl.ds(off, _N // 32)])

    return _sc_unsort


def kernel(vertices, objects, mask):
    vt = vertices.reshape(_N, 18).T.reshape(18, _R, _C)
    ot = objects.reshape(_N, 6).T.reshape(6, _R, _C)
    mf = mask.reshape(_R, _C).astype(jnp.float32)
    rank2d, sidx2d = pl.pallas_call(
        _body,
        out_shape=(
            jax.ShapeDtypeStruct((_R, _C), jnp.int32),
            jax.ShapeDtypeStruct((_R, _C), jnp.int32),
        ),
        out_specs=(
            pl.BlockSpec(memory_space=pltpu.VMEM),
            pl.BlockSpec(memory_space=pltpu.VMEM),
        ),
    )(ot)
    total = pl.pallas_call(
        _reduce_body,
        out_shape=jax.ShapeDtypeStruct((1, 1), jnp.float32),
        out_specs=pl.BlockSpec(memory_space=pltpu.SMEM),
    )(vt, mf)
    groups = _get_sc_unsort()(sidx2d.reshape(_N), rank2d.reshape(_N))
    return groups.reshape(mask.shape), total[0, 0]
